# baseline (speedup 1.0000x reference)
import jax
import jax.numpy as jnp
from jax.experimental import pallas as pl
from jax.experimental.pallas import tpu as pltpu

F32 = jnp.float32

HP, WP = 49, 48
NP = HP * WP
PAD = 56


NF = 102 * 102
XOFF = 320


def _stem_k(xp_ref, mr_ref, w_ref, aw_ref, ab_ref, o_ref, mo_ref,
            xmp_ref, a_ref, h1p_ref, mp_ref):
    m = (mr_ref[...] > 0.7).astype(F32)
    xmp_ref[...] = jnp.zeros_like(xmp_ref)
    xmp_ref[:, pl.ds(XOFF, NF)] = xp_ref[...] * m

    a_ref[...] = jnp.zeros_like(a_ref)
    k = 0
    for dy in range(7):
        for dx in range(7):
            off = XOFF + (dy - 3) * 102 + (dx - 3)
            a_ref[pl.ds(3 * k, 3), :] = xmp_ref[:, pl.ds(off, NF)]
            k += 1

    h1 = jnp.dot(w_ref[...], a_ref[...], preferred_element_type=F32)
    h1 = m * jnp.maximum(h1 * aw_ref[...] + ab_ref[...], 0.0)

    h1p_ref[...] = jnp.zeros_like(h1p_ref)
    h1p_ref[:, pl.ds(0, NF)] = h1
    mp_ref[...] = jnp.zeros_like(mp_ref)
    mp_ref[:, pl.ds(0, NF)] = m

    pmax = h1p_ref[:, pl.ds(0, NF)]
    mmax = mp_ref[:, pl.ds(0, NF)]
    for dy in range(3):
        for dx in range(3):
            if dy == 0 and dx == 0:
                continue
            off = dy * 102 + dx
            pmax = jnp.maximum(pmax, h1p_ref[:, pl.ds(off, NF)])
            mmax = jnp.maximum(mmax, mp_ref[:, pl.ds(off, NF)])
    o_ref[...] = pmax * mmax
    mo_ref[...] = mmax


BF16 = jnp.bfloat16


def _split(a):
    hi = a.astype(BF16)
    lo = (a - hi.astype(F32)).astype(BF16)
    return hi, lo


def _dot2(ah, al, wh, wl):
    return (jnp.dot(ah, wh, preferred_element_type=F32)
            + jnp.dot(ah, wl, preferred_element_type=F32)
            + jnp.dot(al, wh, preferred_element_type=F32))


def _block_k(ds, innerC, *refs):
    if ds:
        (h_ref, m_ref, w1_ref, a1w_ref, a1b_ref, w2_ref, a2w_ref, a2b_ref,
         w3_ref, a3w_ref, a3b_ref, wd_ref, adw_ref, adb_ref,
         o_ref, o1p_ref) = refs
    else:
        (h_ref, m_ref, w1_ref, a1w_ref, a1b_ref, w2_ref, a2w_ref, a2b_ref,
         w3_ref, a3w_ref, a3b_ref, o_ref, o1p_ref) = refs

    m = m_ref[...]
    h = h_ref[...]
    o1 = jnp.dot(h, w1_ref[...], preferred_element_type=F32)
    o1 = m * jnp.maximum(o1 * a1w_ref[...] + a1b_ref[...], 0.0)

    o1p_ref[...] = jnp.zeros((NP + 2 * PAD, innerC), F32)
    o1p_ref[pl.ds(PAD, NP), :] = o1

    acc = jnp.zeros((NP, innerC), F32)
    k = 0
    for dy in (-1, 0, 1):
        for dx in (-1, 0, 1):
            off = PAD + dy * WP + dx
            acc += jnp.dot(o1p_ref[pl.ds(off, NP), :],
                           w2_ref[pl.ds(k * innerC, innerC), :],
                           preferred_element_type=F32)
            k += 1
    o2 = m * jnp.maximum(acc * a2w_ref[...] + a2b_ref[...], 0.0)

    o3 = jnp.dot(o2, w3_ref[...], preferred_element_type=F32)
    o3 = m * (o3 * a3w_ref[...] + a3b_ref[...])

    if ds:
        res = jnp.dot(h, wd_ref[...], preferred_element_type=F32)
        res = m * (res * adw_ref[...] + adb_ref[...])
    else:
        res = h
    o_ref[...] = jnp.maximum(o3 + res, 0.0)


def _gblock_k(innerC, C, h_in_ref, m_ref, w1_ref, a14_ref, w2_ref, w3_ref,
              a32_ref, wd_ref, ad_ref, o_ref, o1p_ref):
    i = pl.program_id(0)

    @pl.when(i == 0)
    def _():
        o_ref[...] = h_in_ref[...]

    m = m_ref[...]
    h = o_ref[...]
    o1 = jnp.dot(h, w1_ref[0], preferred_element_type=F32)
    o1 = m * jnp.maximum(o1 * a14_ref[0, 0:1, :] + a14_ref[0, 1:2, :], 0.0)

    @pl.when(i == 0)
    def _():
        res = jnp.dot(h, wd_ref[...], preferred_element_type=F32)
        o_ref[...] = m * (res * ad_ref[0:1, :] + ad_ref[1:2, :])

    o1p_ref[...] = jnp.zeros((NP + 2 * PAD, innerC), F32)
    o1p_ref[pl.ds(PAD, NP), :] = o1

    acc = jnp.zeros((NP, innerC), F32)
    k = 0
    for dy in (-1, 0, 1):
        for dx in (-1, 0, 1):
            off = PAD + dy * WP + dx
            acc += jnp.dot(o1p_ref[pl.ds(off, NP), :],
                           w2_ref[0, pl.ds(k * innerC, innerC), :],
                           preferred_element_type=F32)
            k += 1
    o2 = m * jnp.maximum(acc * a14_ref[0, 2:3, :] + a14_ref[0, 3:4, :], 0.0)

    o3 = jnp.dot(o2, w3_ref[0], preferred_element_type=F32)
    o3 = m * (o3 * a32_ref[0, 0:1, :] + a32_ref[0, 1:2, :])

    o_ref[...] = jnp.maximum(o3 + o_ref[...], 0.0)


_BLOCKS = [
    (64, 256, 64, True), (256, 256, 64, False), (256, 256, 64, False),
    (256, 512, 128, True), (512, 512, 128, False), (512, 512, 128, False),
    (512, 512, 128, False),
    (512, 1024, 256, True), (1024, 1024, 256, False), (1024, 1024, 256, False),
    (1024, 1024, 256, False), (1024, 1024, 256, False), (1024, 1024, 256, False),
]


def _embed(a):
    return jnp.pad(a, ((1, 1), (0, 1), (0, 0))).reshape(NP, a.shape[-1])


def _row(v):
    return v.reshape(1, -1)


def kernel(x, mask_raw, params):
    it = iter(params)

    w0 = next(it)
    a0w = next(it)
    a0b = next(it)
    xp = jnp.pad(x.reshape(9216, 3).T.reshape(3, 96, 96),
                 ((0, 0), (3, 3), (3, 3))).reshape(3, NF)
    mrp = jnp.pad(mask_raw.reshape(96, 96),
                  ((3, 3), (3, 3))).reshape(1, NF)
    w0m = jnp.pad(w0.reshape(147, 64).T, ((0, 0), (0, 13)))
    hT, mT = pl.pallas_call(
        _stem_k,
        out_shape=(jax.ShapeDtypeStruct((64, NF), F32),
                   jax.ShapeDtypeStruct((1, NF), F32)),
        scratch_shapes=[pltpu.VMEM((3, NF + 2 * XOFF), F32),
                        pltpu.VMEM((160, NF), F32),
                        pltpu.VMEM((64, NF + 256), F32),
                        pltpu.VMEM((1, NF + 256), F32)],
    )(xp, mrp, w0m, a0w.reshape(64, 1), a0b.reshape(64, 1))
    hds = hT.reshape(64, 102, 102)[:, 3:97:2, 3:97:2].reshape(64, 2209)
    mds = mT.reshape(102, 102)[3:97:2, 3:97:2].reshape(1, 2209)
    h = _embed(hds.T.reshape(47, 47, 64))
    m = _embed(mds.T.reshape(47, 47, 1))

    for C, innerC, R in ((256, 64, 3), (512, 128, 4), (1024, 256, 6)):
        inC = h.shape[1]
        w1s, a14s, w2s, w3s, a32s = [], [], [], [], []
        wd = ad = None
        for r in range(R):
            w1 = next(it).reshape(-1, innerC)
            if r == 0:
                w1 = jnp.pad(w1, ((0, C - inC), (0, 0)))
            w1s.append(w1)
            a14 = [next(it), next(it)]
            w2s.append(next(it).reshape(9 * innerC, innerC))
            a14 += [next(it), next(it)]
            a14s.append(jnp.stack(a14))
            w3s.append(next(it).reshape(innerC, C))
            a32s.append(jnp.stack([next(it), next(it)]))
            if r == 0:
                wd = jnp.pad(next(it).reshape(inC, C), ((0, C - inC), (0, 0)))
                ad = jnp.stack([next(it), next(it)])
        w1s = jnp.stack(w1s)
        a14s = jnp.stack(a14s)
        w2s = jnp.stack(w2s)
        w3s = jnp.stack(w3s)
        a32s = jnp.stack(a32s)
        h = jnp.pad(h, ((0, 0), (0, C - inC)))

        def gbody(*refs, _ic=innerC, _C=C):
            _gblock_k(_ic, _C, *refs)

        h = pl.pallas_call(
            gbody,
            grid=(R,),
            in_specs=[
                pl.BlockSpec((NP, C), lambda i: (0, 0)),
                pl.BlockSpec((NP, 1), lambda i: (0, 0)),
                pl.BlockSpec((1, C, innerC), lambda i: (i, 0, 0)),
                pl.BlockSpec((1, 4, innerC), lambda i: (i, 0, 0)),
                pl.BlockSpec((1, 9 * innerC, innerC), lambda i: (i, 0, 0)),
                pl.BlockSpec((1, innerC, C), lambda i: (i, 0, 0)),
                pl.BlockSpec((1, 2, C), lambda i: (i, 0, 0)),
                pl.BlockSpec((C, C), lambda i: (0, 0)),
                pl.BlockSpec((2, C), lambda i: (0, 0)),
            ],
            out_specs=pl.BlockSpec((NP, C), lambda i: (0, 0)),
            out_shape=jax.ShapeDtypeStruct((NP, C), F32),
            scratch_shapes=[pltpu.VMEM((NP + 2 * PAD, innerC), F32)],
            compiler_params=pltpu.CompilerParams(
                dimension_semantics=("arbitrary",)),
        )(h, m, w1s, a14s, w2s, w3s, a32s, wd, ad)

    out = h.reshape(HP, WP, 1024)[1:48, 0:47, :]
    return out.reshape(1, 47, 47, 1024)


# speedup vs baseline: 1.0126x; 1.0126x over previous
import jax
import jax.numpy as jnp
from jax.experimental import pallas as pl
from jax.experimental.pallas import tpu as pltpu

F32 = jnp.float32

HP, WP = 49, 48
NP = HP * WP
PAD = 56


NF = 102 * 102
XOFF = 320


def _stem_k(xp_ref, mr_ref, w_ref, aw_ref, ab_ref, o_ref, mo_ref,
            xmp_ref, a_ref, h1p_ref, mp_ref):
    m = (mr_ref[...] > 0.7).astype(F32)
    xmp_ref[...] = jnp.zeros_like(xmp_ref)
    xmp_ref[:, pl.ds(XOFF, NF)] = xp_ref[...] * m

    a_ref[...] = jnp.zeros_like(a_ref)
    k = 0
    for dy in range(7):
        for dx in range(7):
            off = XOFF + (dy - 3) * 102 + (dx - 3)
            a_ref[pl.ds(3 * k, 3), :] = xmp_ref[:, pl.ds(off, NF)]
            k += 1

    h1 = jnp.dot(w_ref[...], a_ref[...], preferred_element_type=F32)
    h1 = m * jnp.maximum(h1 * aw_ref[...] + ab_ref[...], 0.0)

    h1p_ref[...] = jnp.zeros_like(h1p_ref)
    h1p_ref[:, pl.ds(0, NF)] = h1
    mp_ref[...] = jnp.zeros_like(mp_ref)
    mp_ref[:, pl.ds(0, NF)] = m

    pmax = h1p_ref[:, pl.ds(0, NF)]
    mmax = mp_ref[:, pl.ds(0, NF)]
    for dy in range(3):
        for dx in range(3):
            if dy == 0 and dx == 0:
                continue
            off = dy * 102 + dx
            pmax = jnp.maximum(pmax, h1p_ref[:, pl.ds(off, NF)])
            mmax = jnp.maximum(mmax, mp_ref[:, pl.ds(off, NF)])
    o_ref[...] = pmax * mmax
    mo_ref[...] = mmax


BF16 = jnp.bfloat16


def _split(a):
    hi = a.astype(BF16)
    lo = (a - hi.astype(F32)).astype(BF16)
    return hi, lo


def _dot2(ah, al, wh, wl):
    return (jnp.dot(ah, wh, preferred_element_type=F32)
            + jnp.dot(ah, wl, preferred_element_type=F32)
            + jnp.dot(al, wh, preferred_element_type=F32))


def _block_k(ds, innerC, *refs):
    if ds:
        (h_ref, m_ref, w1_ref, a1w_ref, a1b_ref, w2_ref, a2w_ref, a2b_ref,
         w3_ref, a3w_ref, a3b_ref, wd_ref, adw_ref, adb_ref,
         o_ref, o1p_ref) = refs
    else:
        (h_ref, m_ref, w1_ref, a1w_ref, a1b_ref, w2_ref, a2w_ref, a2b_ref,
         w3_ref, a3w_ref, a3b_ref, o_ref, o1p_ref) = refs

    m = m_ref[...]
    h = h_ref[...]
    o1 = jnp.dot(h, w1_ref[...], preferred_element_type=F32)
    o1 = m * jnp.maximum(o1 * a1w_ref[...] + a1b_ref[...], 0.0)

    o1p_ref[...] = jnp.zeros((NP + 2 * PAD, innerC), F32)
    o1p_ref[pl.ds(PAD, NP), :] = o1

    acc = jnp.zeros((NP, innerC), F32)
    k = 0
    for dy in (-1, 0, 1):
        for dx in (-1, 0, 1):
            off = PAD + dy * WP + dx
            acc += jnp.dot(o1p_ref[pl.ds(off, NP), :],
                           w2_ref[pl.ds(k * innerC, innerC), :],
                           preferred_element_type=F32)
            k += 1
    o2 = m * jnp.maximum(acc * a2w_ref[...] + a2b_ref[...], 0.0)

    o3 = jnp.dot(o2, w3_ref[...], preferred_element_type=F32)
    o3 = m * (o3 * a3w_ref[...] + a3b_ref[...])

    if ds:
        res = jnp.dot(h, wd_ref[...], preferred_element_type=F32)
        res = m * (res * adw_ref[...] + adb_ref[...])
    else:
        res = h
    o_ref[...] = jnp.maximum(o3 + res, 0.0)


def _gblock_k(innerC, C, h_in_ref, m_ref, w1_ref, a14_ref, w2_ref, w3_ref,
              a32_ref, wd_ref, ad_ref, o_ref, o1p_ref):
    i = pl.program_id(0)

    @pl.when(i == 0)
    def _():
        o_ref[...] = h_in_ref[...]

    m = m_ref[...]
    h = o_ref[...]
    hb = h.astype(BF16)
    o1 = jnp.dot(hb, w1_ref[0], preferred_element_type=F32)
    o1 = m * jnp.maximum(o1 * a14_ref[0, 0:1, :] + a14_ref[0, 1:2, :], 0.0)

    @pl.when(i == 0)
    def _():
        res = jnp.dot(hb, wd_ref[...], preferred_element_type=F32)
        o_ref[...] = m * (res * ad_ref[0:1, :] + ad_ref[1:2, :])

    o1p_ref[...] = jnp.zeros((NP + 2 * PAD, innerC), BF16)
    o1p_ref[pl.ds(PAD, NP), :] = o1.astype(BF16)

    acc = jnp.zeros((NP, innerC), F32)
    k = 0
    for dy in (-1, 0, 1):
        for dx in (-1, 0, 1):
            off = PAD + dy * WP + dx
            acc += jnp.dot(o1p_ref[pl.ds(off, NP), :],
                           w2_ref[0, pl.ds(k * innerC, innerC), :],
                           preferred_element_type=F32)
            k += 1
    o2 = m * jnp.maximum(acc * a14_ref[0, 2:3, :] + a14_ref[0, 3:4, :], 0.0)

    o3 = jnp.dot(o2.astype(BF16), w3_ref[0], preferred_element_type=F32)
    o3 = m * (o3 * a32_ref[0, 0:1, :] + a32_ref[0, 1:2, :])

    o_ref[...] = jnp.maximum(o3 + o_ref[...], 0.0)


_BLOCKS = [
    (64, 256, 64, True), (256, 256, 64, False), (256, 256, 64, False),
    (256, 512, 128, True), (512, 512, 128, False), (512, 512, 128, False),
    (512, 512, 128, False),
    (512, 1024, 256, True), (1024, 1024, 256, False), (1024, 1024, 256, False),
    (1024, 1024, 256, False), (1024, 1024, 256, False), (1024, 1024, 256, False),
]


def _embed(a):
    return jnp.pad(a, ((1, 1), (0, 1), (0, 0))).reshape(NP, a.shape[-1])


def _row(v):
    return v.reshape(1, -1)


def kernel(x, mask_raw, params):
    it = iter(params)

    w0 = next(it)
    a0w = next(it)
    a0b = next(it)
    xp = jnp.pad(x.reshape(9216, 3).T.reshape(3, 96, 96),
                 ((0, 0), (3, 3), (3, 3))).reshape(3, NF)
    mrp = jnp.pad(mask_raw.reshape(96, 96),
                  ((3, 3), (3, 3))).reshape(1, NF)
    w0m = jnp.pad(w0.reshape(147, 64).T, ((0, 0), (0, 13)))
    hT, mT = pl.pallas_call(
        _stem_k,
        out_shape=(jax.ShapeDtypeStruct((64, NF), F32),
                   jax.ShapeDtypeStruct((1, NF), F32)),
        scratch_shapes=[pltpu.VMEM((3, NF + 2 * XOFF), F32),
                        pltpu.VMEM((160, NF), F32),
                        pltpu.VMEM((64, NF + 256), F32),
                        pltpu.VMEM((1, NF + 256), F32)],
    )(xp, mrp, w0m, a0w.reshape(64, 1), a0b.reshape(64, 1))
    hds = hT.reshape(64, 102, 102)[:, 3:97:2, 3:97:2].reshape(64, 2209)
    mds = mT.reshape(102, 102)[3:97:2, 3:97:2].reshape(1, 2209)
    h = _embed(hds.T.reshape(47, 47, 64))
    m = _embed(mds.T.reshape(47, 47, 1))

    for C, innerC, R in ((256, 64, 3), (512, 128, 4), (1024, 256, 6)):
        inC = h.shape[1]
        w1s, a14s, w2s, w3s, a32s = [], [], [], [], []
        wd = ad = None
        for r in range(R):
            w1 = next(it).reshape(-1, innerC)
            if r == 0:
                w1 = jnp.pad(w1, ((0, C - inC), (0, 0)))
            w1s.append(w1)
            a14 = [next(it), next(it)]
            w2s.append(next(it).reshape(9 * innerC, innerC))
            a14 += [next(it), next(it)]
            a14s.append(jnp.stack(a14))
            w3s.append(next(it).reshape(innerC, C))
            a32s.append(jnp.stack([next(it), next(it)]))
            if r == 0:
                wd = jnp.pad(next(it).reshape(inC, C), ((0, C - inC), (0, 0)))
                ad = jnp.stack([next(it), next(it)])
        w1s = jnp.stack(w1s).astype(BF16)
        a14s = jnp.stack(a14s)
        w2s = jnp.stack(w2s).astype(BF16)
        w3s = jnp.stack(w3s).astype(BF16)
        a32s = jnp.stack(a32s)
        wd = wd.astype(BF16)
        h = jnp.pad(h, ((0, 0), (0, C - inC)))

        def gbody(*refs, _ic=innerC, _C=C):
            _gblock_k(_ic, _C, *refs)

        h = pl.pallas_call(
            gbody,
            grid=(R,),
            in_specs=[
                pl.BlockSpec((NP, C), lambda i: (0, 0)),
                pl.BlockSpec((NP, 1), lambda i: (0, 0)),
                pl.BlockSpec((1, C, innerC), lambda i: (i, 0, 0)),
                pl.BlockSpec((1, 4, innerC), lambda i: (i, 0, 0)),
                pl.BlockSpec((1, 9 * innerC, innerC), lambda i: (i, 0, 0)),
                pl.BlockSpec((1, innerC, C), lambda i: (i, 0, 0)),
                pl.BlockSpec((1, 2, C), lambda i: (i, 0, 0)),
                pl.BlockSpec((C, C), lambda i: (0, 0)),
                pl.BlockSpec((2, C), lambda i: (0, 0)),
            ],
            out_specs=pl.BlockSpec((NP, C), lambda i: (0, 0)),
            out_shape=jax.ShapeDtypeStruct((NP, C), F32),
            scratch_shapes=[pltpu.VMEM((NP + 2 * PAD, innerC), BF16)],
            compiler_params=pltpu.CompilerParams(
                dimension_semantics=("arbitrary",)),
        )(h, m, w1s, a14s, w2s, w3s, a32s, wd, ad)

    out = h.reshape(HP, WP, 1024)[1:48, 0:47, :]
    return out.reshape(1, 47, 47, 1024)


# speedup vs baseline: 1.0832x; 1.0697x over previous
import jax
import jax.numpy as jnp
from jax.experimental import pallas as pl
from jax.experimental.pallas import tpu as pltpu

F32 = jnp.float32

HP, WP = 49, 48
NP = HP * WP
PAD = 56


NF = 102 * 102
XOFF = 320


def _stem_k(xp_ref, mr_ref, w_ref, aw_ref, ab_ref, o_ref, mo_ref,
            xmp_ref, a_ref, h1p_ref, mp_ref):
    m = (mr_ref[...] > 0.7).astype(F32)
    xmp_ref[...] = jnp.zeros_like(xmp_ref)
    xmp_ref[:, pl.ds(XOFF, NF)] = xp_ref[...] * m

    a_ref[...] = jnp.zeros_like(a_ref)
    k = 0
    for dy in range(7):
        for dx in range(7):
            off = XOFF + (dy - 3) * 102 + (dx - 3)
            a_ref[pl.ds(3 * k, 3), :] = xmp_ref[:, pl.ds(off, NF)]
            k += 1

    h1 = jnp.dot(w_ref[...], a_ref[...], preferred_element_type=F32)
    h1 = m * jnp.maximum(h1 * aw_ref[...] + ab_ref[...], 0.0)

    h1p_ref[...] = jnp.zeros_like(h1p_ref)
    h1p_ref[:, pl.ds(0, NF)] = h1
    mp_ref[...] = jnp.zeros_like(mp_ref)
    mp_ref[:, pl.ds(0, NF)] = m

    pmax = h1p_ref[:, pl.ds(0, NF)]
    mmax = mp_ref[:, pl.ds(0, NF)]
    for dy in range(3):
        for dx in range(3):
            if dy == 0 and dx == 0:
                continue
            off = dy * 102 + dx
            pmax = jnp.maximum(pmax, h1p_ref[:, pl.ds(off, NF)])
            mmax = jnp.maximum(mmax, mp_ref[:, pl.ds(off, NF)])
    o_ref[...] = pmax * mmax
    mo_ref[...] = mmax


BF16 = jnp.bfloat16


def _split(a):
    hi = a.astype(BF16)
    lo = (a - hi.astype(F32)).astype(BF16)
    return hi, lo


def _dot2(ah, al, wh, wl):
    return (jnp.dot(ah, wh, preferred_element_type=F32)
            + jnp.dot(ah, wl, preferred_element_type=F32)
            + jnp.dot(al, wh, preferred_element_type=F32))


def _block_k(ds, innerC, *refs):
    if ds:
        (h_ref, m_ref, w1_ref, a1w_ref, a1b_ref, w2_ref, a2w_ref, a2b_ref,
         w3_ref, a3w_ref, a3b_ref, wd_ref, adw_ref, adb_ref,
         o_ref, o1p_ref) = refs
    else:
        (h_ref, m_ref, w1_ref, a1w_ref, a1b_ref, w2_ref, a2w_ref, a2b_ref,
         w3_ref, a3w_ref, a3b_ref, o_ref, o1p_ref) = refs

    m = m_ref[...]
    h = h_ref[...]
    o1 = jnp.dot(h, w1_ref[...], preferred_element_type=F32)
    o1 = m * jnp.maximum(o1 * a1w_ref[...] + a1b_ref[...], 0.0)

    o1p_ref[...] = jnp.zeros((NP + 2 * PAD, innerC), F32)
    o1p_ref[pl.ds(PAD, NP), :] = o1

    acc = jnp.zeros((NP, innerC), F32)
    k = 0
    for dy in (-1, 0, 1):
        for dx in (-1, 0, 1):
            off = PAD + dy * WP + dx
            acc += jnp.dot(o1p_ref[pl.ds(off, NP), :],
                           w2_ref[pl.ds(k * innerC, innerC), :],
                           preferred_element_type=F32)
            k += 1
    o2 = m * jnp.maximum(acc * a2w_ref[...] + a2b_ref[...], 0.0)

    o3 = jnp.dot(o2, w3_ref[...], preferred_element_type=F32)
    o3 = m * (o3 * a3w_ref[...] + a3b_ref[...])

    if ds:
        res = jnp.dot(h, wd_ref[...], preferred_element_type=F32)
        res = m * (res * adw_ref[...] + adb_ref[...])
    else:
        res = h
    o_ref[...] = jnp.maximum(o3 + res, 0.0)


def _gblock_k(innerC, C, inC, h_in_ref, m_ref, w1_ref, a14_ref, w2_ref,
              w3_ref, a32_ref, wd_ref, ad_ref, o_ref, o1p_ref):
    i = pl.program_id(0)

    @pl.when(i == 0)
    def _():
        o_ref[...] = jnp.concatenate(
            [h_in_ref[...], jnp.zeros((NP, C - inC), F32)], axis=1)

    m = m_ref[...]
    h = o_ref[...]
    hb = h.astype(BF16)
    o1 = jnp.dot(hb, w1_ref[0], preferred_element_type=F32)
    o1 = m * jnp.maximum(o1 * a14_ref[0, 0:1, :] + a14_ref[0, 1:2, :], 0.0)

    @pl.when(i == 0)
    def _():
        res = jnp.dot(hb, wd_ref[...], preferred_element_type=F32)
        o_ref[...] = m * (res * ad_ref[0:1, :] + ad_ref[1:2, :])

    o1p_ref[...] = jnp.zeros((NP + 2 * PAD, innerC), BF16)
    o1p_ref[pl.ds(PAD, NP), :] = o1.astype(BF16)

    acc = jnp.zeros((NP, innerC), F32)
    k = 0
    for dy in (-1, 0, 1):
        for dx in (-1, 0, 1):
            off = PAD + dy * WP + dx
            acc += jnp.dot(o1p_ref[pl.ds(off, NP), :],
                           w2_ref[0, pl.ds(k * innerC, innerC), :],
                           preferred_element_type=F32)
            k += 1
    o2 = m * jnp.maximum(acc * a14_ref[0, 2:3, :] + a14_ref[0, 3:4, :], 0.0)

    o3 = jnp.dot(o2.astype(BF16), w3_ref[0], preferred_element_type=F32)
    o3 = m * (o3 * a32_ref[0, 0:1, :] + a32_ref[0, 1:2, :])

    o_ref[...] = jnp.maximum(o3 + o_ref[...], 0.0)


_BLOCKS = [
    (64, 256, 64, True), (256, 256, 64, False), (256, 256, 64, False),
    (256, 512, 128, True), (512, 512, 128, False), (512, 512, 128, False),
    (512, 512, 128, False),
    (512, 1024, 256, True), (1024, 1024, 256, False), (1024, 1024, 256, False),
    (1024, 1024, 256, False), (1024, 1024, 256, False), (1024, 1024, 256, False),
]


def _embed(a):
    return jnp.pad(a, ((1, 1), (0, 1), (0, 0))).reshape(NP, a.shape[-1])


def _row(v):
    return v.reshape(1, -1)


def kernel(x, mask_raw, params):
    it = iter(params)

    w0 = next(it)
    a0w = next(it)
    a0b = next(it)
    xp = jnp.pad(x.reshape(9216, 3).T.reshape(3, 96, 96),
                 ((0, 0), (3, 3), (3, 3))).reshape(3, NF)
    mrp = jnp.pad(mask_raw.reshape(96, 96),
                  ((3, 3), (3, 3))).reshape(1, NF)
    w0m = jnp.pad(w0.reshape(147, 64).T, ((0, 0), (0, 13)))
    hT, mT = pl.pallas_call(
        _stem_k,
        out_shape=(jax.ShapeDtypeStruct((64, NF), F32),
                   jax.ShapeDtypeStruct((1, NF), F32)),
        scratch_shapes=[pltpu.VMEM((3, NF + 2 * XOFF), F32),
                        pltpu.VMEM((160, NF), F32),
                        pltpu.VMEM((64, NF + 256), F32),
                        pltpu.VMEM((1, NF + 256), F32)],
    )(xp, mrp, w0m, a0w.reshape(64, 1), a0b.reshape(64, 1))
    hds = hT.reshape(64, 102, 102)[:, 3:97:2, 3:97:2].reshape(64, 2209)
    mds = mT.reshape(102, 102)[3:97:2, 3:97:2].reshape(1, 2209)
    h = _embed(hds.T.reshape(47, 47, 64))
    m = _embed(mds.T.reshape(47, 47, 1))

    for C, innerC, R in ((256, 64, 3), (512, 128, 4), (1024, 256, 6)):
        inC = h.shape[1]
        w1s, a14s, w2s, w3s, a32s = [], [], [], [], []
        wd = ad = None
        for r in range(R):
            w1 = next(it).reshape(-1, innerC)
            if r == 0:
                w1 = jnp.pad(w1, ((0, C - inC), (0, 0)))
            w1s.append(w1)
            a14 = [next(it), next(it)]
            w2s.append(next(it).reshape(9 * innerC, innerC))
            a14 += [next(it), next(it)]
            a14s.append(jnp.stack(a14))
            w3s.append(next(it).reshape(innerC, C))
            a32s.append(jnp.stack([next(it), next(it)]))
            if r == 0:
                wd = jnp.pad(next(it).reshape(inC, C), ((0, C - inC), (0, 0)))
                ad = jnp.stack([next(it), next(it)])
        w1s = jnp.stack(w1s).astype(BF16)
        a14s = jnp.stack(a14s)
        w2s = jnp.stack(w2s).astype(BF16)
        w3s = jnp.stack(w3s).astype(BF16)
        a32s = jnp.stack(a32s)
        wd = wd.astype(BF16)

        def gbody(*refs, _ic=innerC, _C=C, _inC=inC):
            _gblock_k(_ic, _C, _inC, *refs)

        h = pl.pallas_call(
            gbody,
            grid=(R,),
            in_specs=[
                pl.BlockSpec((NP, inC), lambda i: (0, 0)),
                pl.BlockSpec((NP, 1), lambda i: (0, 0)),
                pl.BlockSpec((1, C, innerC), lambda i: (i, 0, 0)),
                pl.BlockSpec((1, 4, innerC), lambda i: (i, 0, 0)),
                pl.BlockSpec((1, 9 * innerC, innerC), lambda i: (i, 0, 0)),
                pl.BlockSpec((1, innerC, C), lambda i: (i, 0, 0)),
                pl.BlockSpec((1, 2, C), lambda i: (i, 0, 0)),
                pl.BlockSpec((C, C), lambda i: (0, 0)),
                pl.BlockSpec((2, C), lambda i: (0, 0)),
            ],
            out_specs=pl.BlockSpec((NP, C), lambda i: (0, 0)),
            out_shape=jax.ShapeDtypeStruct((NP, C), F32),
            scratch_shapes=[pltpu.VMEM((NP + 2 * PAD, innerC), BF16)],
            compiler_params=pltpu.CompilerParams(
                dimension_semantics=("arbitrary",)),
        )(h, m, w1s, a14s, w2s, w3s, a32s, wd, ad)

    out = h.reshape(HP, WP, 1024)[1:48, 0:47, :]
    return out.reshape(1, 47, 47, 1024)
